# baseline (speedup 1.0000x reference)
import functools

import jax
import jax.numpy as jnp
from jax.experimental import pallas as pl
from jax.experimental.pallas import tpu as pltpu

E = 8
TOP_K = 2
HIDDEN = 1024
FFN = 1408
ALPHA = 0.001
SCALING = 1.0

BLK = 128
NBLK = 40
NPAD = NBLK * BLK


def _gate_kernel(x_ref, gw_ref, i1_ref, i2_ref, w0_ref, w1_ref, laux_ref):
    x = x_ref[...]
    gw = gw_ref[...]
    logits = jax.lax.dot_general(x, gw, (((1,), (1,)), ((), ())),
                                 preferred_element_type=jnp.float32)
    m = jnp.max(logits, axis=-1, keepdims=True)
    ex = jnp.exp(logits - m)
    s = ex / jnp.sum(ex, axis=-1, keepdims=True)
    iota = jax.lax.broadcasted_iota(jnp.int32, s.shape, 1)
    m1 = jnp.max(s, axis=-1, keepdims=True)
    i1 = jnp.min(jnp.where(s == m1, iota, E), axis=-1, keepdims=True)
    s2 = jnp.where(iota == i1, -1.0, s)
    m2 = jnp.max(s2, axis=-1, keepdims=True)
    i2 = jnp.min(jnp.where(s2 == m2, iota, E), axis=-1, keepdims=True)
    denom = m1 + m2 + 1e-20
    w0 = (m1 / denom) * SCALING
    w1 = (m2 / denom) * SCALING
    counts = jnp.sum((iota == i1).astype(jnp.float32)
                     + (iota == i2).astype(jnp.float32), axis=0)
    ssum = jnp.sum(s, axis=0)
    T = x.shape[0]
    laux = jnp.sum(ssum * counts) * (ALPHA * E / (T * TOP_K * T))
    i1_ref[...] = i1[:, 0]
    i2_ref[...] = i2[:, 0]
    w0_ref[...] = w0[:, 0]
    w1_ref[...] = w1[:, 0]
    laux_ref[...] = laux.reshape(1, 1)


def _expert_kernel(be_ref, xs_ref, w1_ref, w2_ref, ys_ref):
    x = xs_ref[...].astype(jnp.bfloat16)
    mid = jnp.dot(x, w1_ref[0], preferred_element_type=jnp.float32)
    g = mid[:, :FFN]
    u = mid[:, FFN:]
    act = (g * jax.lax.logistic(g) * u).astype(jnp.bfloat16)
    ys_ref[...] = jnp.dot(act, w2_ref[0], preferred_element_type=jnp.float32)


def kernel(hidden_states, gate_weight, w1, w2):
    seq, b, h = hidden_states.shape
    T = seq * b
    x = hidden_states.reshape(T, h)

    i1, i2, wt0, wt1, laux = pl.pallas_call(
        _gate_kernel,
        out_shape=(
            jax.ShapeDtypeStruct((T,), jnp.int32),
            jax.ShapeDtypeStruct((T,), jnp.int32),
            jax.ShapeDtypeStruct((T,), jnp.float32),
            jax.ShapeDtypeStruct((T,), jnp.float32),
            jax.ShapeDtypeStruct((1, 1), jnp.float32),
        ),
    )(x, gate_weight)

    er = jnp.arange(E, dtype=jnp.int32)[None, :]
    oh0 = (i1[:, None] == er).astype(jnp.int32)
    oh1 = (i2[:, None] == er).astype(jnp.int32)
    c0 = jnp.cumsum(oh0, axis=0)
    c1 = jnp.cumsum(oh1, axis=0)
    counts0 = c0[-1]
    counts = counts0 + c1[-1]
    rank0 = jnp.take_along_axis(c0, i1[:, None], axis=1)[:, 0] - 1
    rank1 = counts0[i2] + jnp.take_along_axis(c1, i2[:, None], axis=1)[:, 0] - 1
    padded = ((counts + BLK - 1) // BLK) * BLK
    offs = jnp.concatenate([jnp.zeros(1, jnp.int32), jnp.cumsum(padded)[:-1]])
    pos0 = offs[i1] + rank0
    pos1 = offs[i2] + rank1
    cumblk = jnp.cumsum(padded // BLK)
    be = jnp.sum((jnp.arange(NBLK, dtype=jnp.int32)[:, None]
                  >= cumblk[None, :]).astype(jnp.int32), axis=1)
    be = jnp.minimum(be, E - 1).astype(jnp.int32)

    tok = jnp.arange(T, dtype=jnp.int32)
    src_tok = jnp.zeros(NPAD, jnp.int32).at[pos0].set(tok).at[pos1].set(tok)
    xs = x[src_tok]

    ys = pl.pallas_call(
        _expert_kernel,
        grid_spec=pltpu.PrefetchScalarGridSpec(
            num_scalar_prefetch=1,
            grid=(NBLK,),
            in_specs=[
                pl.BlockSpec((BLK, HIDDEN), lambda i, be_r: (i, 0)),
                pl.BlockSpec((1, HIDDEN, 2 * FFN), lambda i, be_r: (be_r[i], 0, 0)),
                pl.BlockSpec((1, FFN, HIDDEN), lambda i, be_r: (be_r[i], 0, 0)),
            ],
            out_specs=pl.BlockSpec((BLK, HIDDEN), lambda i, be_r: (i, 0)),
        ),
        out_shape=jax.ShapeDtypeStruct((NPAD, HIDDEN), jnp.float32),
    )(be, xs, w1.astype(jnp.bfloat16), w2.astype(jnp.bfloat16))

    out = wt0[:, None] * ys[pos0] + wt1[:, None] * ys[pos1]
    out = out.reshape(seq, b, h)
    return out, laux[0, 0]


# speedup vs baseline: 1.2657x; 1.2657x over previous
import functools

import jax
import jax.numpy as jnp
from jax.experimental import pallas as pl
from jax.experimental.pallas import tpu as pltpu

E = 8
TOP_K = 2
HIDDEN = 1024
FFN = 1408
ALPHA = 0.001
SCALING = 1.0

BLK = 128
NBLK = 40
NPAD = NBLK * BLK


def _gate_kernel(x_ref, gw_ref, pos0_ref, pos1_ref, wt0_ref, wt1_ref,
                 be_ref, laux_ref):
    x = x_ref[...]
    gw = gw_ref[...]
    T = x.shape[0]
    logits = jax.lax.dot_general(x, gw, (((1,), (1,)), ((), ())),
                                 preferred_element_type=jnp.float32)
    m = jnp.max(logits, axis=-1, keepdims=True)
    ex = jnp.exp(logits - m)
    s = ex / jnp.sum(ex, axis=-1, keepdims=True)
    iota = jax.lax.broadcasted_iota(jnp.int32, s.shape, 1)
    m1 = jnp.max(s, axis=-1, keepdims=True)
    i1 = jnp.min(jnp.where(s == m1, iota, E), axis=-1, keepdims=True)
    s2 = jnp.where(iota == i1, -1.0, s)
    m2 = jnp.max(s2, axis=-1, keepdims=True)
    i2 = jnp.min(jnp.where(s2 == m2, iota, E), axis=-1, keepdims=True)
    denom = m1 + m2 + 1e-20
    wt0_ref[...] = ((m1 / denom) * SCALING)[:, 0]
    wt1_ref[...] = ((m2 / denom) * SCALING)[:, 0]

    oh0 = (iota == i1).astype(jnp.float32)
    oh1 = (iota == i2).astype(jnp.float32)

    r = jax.lax.broadcasted_iota(jnp.int32, (T, T), 0)
    c = jax.lax.broadcasted_iota(jnp.int32, (T, T), 1)
    L = (r >= c).astype(jnp.float32)
    c0 = jax.lax.dot_general(L, oh0, (((1,), (0,)), ((), ())),
                             preferred_element_type=jnp.float32)
    c1 = jax.lax.dot_general(L, oh1, (((1,), (0,)), ((), ())),
                             preferred_element_type=jnp.float32)
    counts0 = jnp.sum(oh0, axis=0)
    counts1 = jnp.sum(oh1, axis=0)
    counts = counts0 + counts1
    rank0 = jnp.sum(c0 * oh0, axis=-1) - 1.0
    rank1 = jnp.sum((c1 + counts0[None, :]) * oh1, axis=-1) - 1.0

    counts_i = counts.astype(jnp.int32)
    padded = ((counts_i + (BLK - 1)) // BLK) * BLK
    er = jax.lax.broadcasted_iota(jnp.int32, (E, E), 0)
    ec = jax.lax.broadcasted_iota(jnp.int32, (E, E), 1)
    offs = jnp.sum(jnp.where(ec < er, padded[None, :], 0), axis=1)
    cumblk = jnp.sum(jnp.where(ec <= er, padded[None, :] // BLK, 0), axis=1)

    pos0_ref[...] = (jnp.sum(oh0 * offs[None, :].astype(jnp.float32), axis=-1)
                     + rank0).astype(jnp.int32)
    pos1_ref[...] = (jnp.sum(oh1 * offs[None, :].astype(jnp.float32), axis=-1)
                     + rank1).astype(jnp.int32)

    bi = jax.lax.broadcasted_iota(jnp.int32, (NBLK, E), 0)
    be = jnp.sum((bi >= cumblk[None, :]).astype(jnp.int32), axis=1)
    be_ref[...] = jnp.minimum(be, E - 1)

    ssum = jnp.sum(s, axis=0)
    laux = jnp.sum(ssum * counts) * (ALPHA * E / (T * TOP_K * T))
    laux_ref[...] = laux.reshape(1, 1)


def _expert_kernel(be_ref, xs_ref, w1_ref, w2_ref, ys_ref):
    x = xs_ref[...]
    mid = jnp.dot(x, w1_ref[0], preferred_element_type=jnp.float32)
    g = mid[:, :FFN]
    u = mid[:, FFN:]
    act = g * jax.lax.logistic(g) * u
    ys_ref[...] = jnp.dot(act, w2_ref[0], preferred_element_type=jnp.float32)


def kernel(hidden_states, gate_weight, w1, w2):
    seq, b, h = hidden_states.shape
    T = seq * b
    x = hidden_states.reshape(T, h)

    pos0, pos1, wt0, wt1, be, laux = pl.pallas_call(
        _gate_kernel,
        out_shape=(
            jax.ShapeDtypeStruct((T,), jnp.int32),
            jax.ShapeDtypeStruct((T,), jnp.int32),
            jax.ShapeDtypeStruct((T,), jnp.float32),
            jax.ShapeDtypeStruct((T,), jnp.float32),
            jax.ShapeDtypeStruct((NBLK,), jnp.int32),
            jax.ShapeDtypeStruct((1, 1), jnp.float32),
        ),
    )(x, gate_weight)

    tok = jnp.arange(T, dtype=jnp.int32)
    src_tok = jnp.zeros(NPAD, jnp.int32).at[pos0].set(tok).at[pos1].set(tok)
    xs = x[src_tok]

    ys = pl.pallas_call(
        _expert_kernel,
        grid_spec=pltpu.PrefetchScalarGridSpec(
            num_scalar_prefetch=1,
            grid=(NBLK,),
            in_specs=[
                pl.BlockSpec((BLK, HIDDEN), lambda i, be_r: (i, 0)),
                pl.BlockSpec((1, HIDDEN, 2 * FFN), lambda i, be_r: (be_r[i], 0, 0)),
                pl.BlockSpec((1, FFN, HIDDEN), lambda i, be_r: (be_r[i], 0, 0)),
            ],
            out_specs=pl.BlockSpec((BLK, HIDDEN), lambda i, be_r: (i, 0)),
        ),
        out_shape=jax.ShapeDtypeStruct((NPAD, HIDDEN), jnp.float32),
    )(be, xs, w1, w2)

    out = wt0[:, None] * ys[pos0] + wt1[:, None] * ys[pos1]
    out = out.reshape(seq, b, h)
    return out, laux[0, 0]
